# Initial kernel scaffold; baseline (speedup 1.0000x reference)
import jax
import jax.numpy as jnp
from jax.experimental import pallas as pl


def kernel(vecs, pred_to_img, W, b):
    raise NotImplementedError("write your pallas kernel here")


# speedup vs baseline: 3.6763x; 3.6763x over previous
import jax
import jax.numpy as jnp
from jax import lax
from jax.experimental import pallas as pl
from jax.experimental.pallas import tpu as pltpu
from jax.experimental.pallas import tpu_sc as plsc

NC = 2
NS = 16
NW = NC * NS


def _sc_segment_sum(vecs, pred_to_img, n_seg):
    O, D = vecs.shape
    P = O // NW
    C = 80
    ITERS = P // C
    assert P % C == 0 and O % NW == 0
    STRIPE = (n_seg // NS) // 8 * 8
    TAIL = n_seg - NS * STRIPE
    ZR = 24
    assert STRIPE % ZR == 0 and TAIL <= ZR and TAIL % 8 == 0

    mesh = plsc.VectorSubcoreMesh(core_axis_name="c", subcore_axis_name="s")

    def body(vecs_hbm, idx_hbm, partial_hbm, acc, rowbuf, idxbuf, zbuf):
        c = lax.axis_index("c")
        s = lax.axis_index("s")
        wid = c * NS + s

        for r in range(ZR):
            for j in range(D // 16):
                zbuf[r, pl.ds(j * 16, 16)] = jnp.zeros((16,), jnp.float32)
        stripe = s * STRIPE
        for k in range(STRIPE // ZR):
            pltpu.sync_copy(zbuf, acc.at[pl.ds(stripe + k * ZR, ZR)])

        @pl.when(s == NS - 1)
        def _():
            pltpu.sync_copy(zbuf.at[pl.ds(0, TAIL)],
                            acc.at[pl.ds(NS * STRIPE, TAIL)])

        plsc.subcore_barrier()

        row_base = wid * P

        def step(i, carry):
            base = row_base + i * C
            pltpu.sync_copy(idx_hbm.at[pl.ds(base, C)], idxbuf)
            pltpu.sync_copy(vecs_hbm.at[pl.ds(base, C)], rowbuf)
            pltpu.sync_copy(rowbuf, acc.at[idxbuf], add=True)
            return carry

        lax.fori_loop(0, ITERS, step, 0)
        plsc.subcore_barrier()

        pltpu.sync_copy(acc.at[pl.ds(stripe, STRIPE)],
                        partial_hbm.at[c, pl.ds(stripe, STRIPE)])

        @pl.when(s == NS - 1)
        def _():
            pltpu.sync_copy(acc.at[pl.ds(NS * STRIPE, TAIL)],
                            partial_hbm.at[c, pl.ds(NS * STRIPE, TAIL)])

    f = pl.kernel(
        body,
        out_type=jax.ShapeDtypeStruct((NC, n_seg, D), jnp.float32),
        mesh=mesh,
        scratch_types=[
            pltpu.VMEM_SHARED((n_seg, D), jnp.float32),
            pltpu.VMEM((C, D), jnp.float32),
            pltpu.VMEM((C,), jnp.int32),
            pltpu.VMEM((ZR, D), jnp.float32),
        ],
    )
    return f(vecs, pred_to_img)


def _tc_merge_fc(partial, W, b):
    NCp, N, D = partial.shape
    out_dim = W.shape[0]
    BLK = 1000
    grid = N // BLK

    def body(p_ref, w_ref, b_ref, ctx_ref, emb_ref):
        ctx = p_ref[0] + p_ref[1]
        ctx_ref[...] = ctx
        emb = lax.dot_general(ctx, w_ref[...], (((1,), (1,)), ((), ())),
                              preferred_element_type=jnp.float32)
        emb_ref[...] = emb + b_ref[...]

    return pl.pallas_call(
        body,
        grid=(grid,),
        in_specs=[
            pl.BlockSpec((NCp, BLK, D), lambda i: (0, i, 0)),
            pl.BlockSpec((out_dim, D), lambda i: (0, 0)),
            pl.BlockSpec((1, out_dim), lambda i: (0, 0)),
        ],
        out_specs=[
            pl.BlockSpec((BLK, D), lambda i: (i, 0)),
            pl.BlockSpec((BLK, out_dim), lambda i: (i, 0)),
        ],
        out_shape=[
            jax.ShapeDtypeStruct((N, D), jnp.float32),
            jax.ShapeDtypeStruct((N, out_dim), jnp.float32),
        ],
    )(partial, W, b.reshape(1, out_dim))


def kernel(vecs, pred_to_img, W, b):
    n_seg = 10000
    partial = _sc_segment_sum(vecs, pred_to_img, n_seg)
    context, embedding = _tc_merge_fc(partial, W, b)
    return (context, embedding)
